# baseline (speedup 1.0000x reference)
import jax
import jax.numpy as jnp
from jax import lax
from jax.experimental import pallas as pl
from jax.experimental.pallas import tpu as pltpu
from jax.experimental.pallas import tpu_sc as plsc

VOCAB = 100000
DIM = 128
B = 4096
L = 50

NC = 2
NS = 16
NW = NC * NS

BPW = B // NW
CHUNK = 64
NBUF = 10
LPR = NBUF // 2
ROUNDS = L // LPR


def _body(words_hbm, pre_hbm, trn_hbm, out_hbm, idx_v, *rest):
    bufs = rest[0:NBUF]
    semg = rest[NBUF : 2 * NBUF]
    sema = rest[2 * NBUF : 3 * NBUF]
    semo = rest[3 * NBUF : 4 * NBUF]

    wid = lax.axis_index("s") * NC + lax.axis_index("c")
    brow0 = wid * BPW

    pltpu.sync_copy(words_hbm.at[:, pl.ds(brow0, BPW)], idx_v)

    def idx_sl(l, b):
        return idx_v.at[l, pl.ds((b % 2) * CHUNK, CHUNK)]

    def fire_g1(l, b):
        pltpu.async_copy(pre_hbm.at[idx_sl(l, b)], bufs[b], semg[b])

    def wait_g1(l, b):
        pltpu.make_async_copy(pre_hbm.at[idx_sl(l, b)], bufs[b], semg[b]).wait()

    def process(l, b):
        cp = pltpu.async_copy(trn_hbm.at[idx_sl(l, b)], bufs[b], sema[b], add=True)
        cp.wait()
        pltpu.async_copy(
            bufs[b], out_hbm.at[l, pl.ds(brow0 + (b % 2) * CHUNK, CHUNK)], semo[b]
        )

    def wait_out(b):
        pltpu.make_async_copy(
            bufs[b], out_hbm.at[0, pl.ds(brow0, CHUNK)], semo[b]
        ).wait()

    for b in range(NBUF):
        fire_g1(b // 2, b)

    def round_body(g, carry):
        l0 = g * LPR
        for b in range(NBUF):
            wait_g1(l0 + b // 2, b)
            process(l0 + b // 2, b)
        for b in range(NBUF):
            wait_out(b)
            fire_g1(l0 + b // 2 + LPR, b)
        return carry

    lax.fori_loop(0, ROUNDS - 1, round_body, 0)

    l0 = (ROUNDS - 1) * LPR
    for b in range(NBUF):
        wait_g1(l0 + b // 2, b)
        process(l0 + b // 2, b)
    for b in range(NBUF):
        wait_out(b)


@jax.jit
def _run(words, pre, trn):
    mesh = plsc.VectorSubcoreMesh(
        core_axis_name="c", subcore_axis_name="s", num_cores=NC, num_subcores=NS
    )
    f = pl.kernel(
        _body,
        out_type=jax.ShapeDtypeStruct((L, B, DIM), jnp.float32),
        mesh=mesh,
        scratch_types=(
            [pltpu.VMEM((L, BPW), jnp.int32)]
            + [pltpu.VMEM((CHUNK, DIM), jnp.float32) for _ in range(NBUF)]

            + [pltpu.SemaphoreType.DMA for _ in range(3 * NBUF)]
        ),
    )
    out_lbd = f(jnp.transpose(words), pre, trn)
    return jnp.transpose(out_lbd, (1, 0, 2))


def kernel(words, pretrained_weight, trainable_weight):
    return _run(words, pretrained_weight, trainable_weight)


# speedup vs baseline: 1.3945x; 1.3945x over previous
import jax
import jax.numpy as jnp
from jax import lax
from jax.experimental import pallas as pl
from jax.experimental.pallas import tpu as pltpu
from jax.experimental.pallas import tpu_sc as plsc

VOCAB = 100000
DIM = 128
B = 4096
L = 50

NC = 2
NS = 16
NW = NC * NS

BPW = B // NW
CHUNK = BPW
NBUF = 5
ROUNDS = L // NBUF


def _body(words_hbm, pre_hbm, trn_hbm, out_hbm, idx_v, *rest):
    bufs = rest[0:NBUF]
    semg = rest[NBUF : 2 * NBUF]
    sema = rest[2 * NBUF : 3 * NBUF]
    semo = rest[3 * NBUF : 4 * NBUF]

    wid = lax.axis_index("s") * NC + lax.axis_index("c")
    brow0 = wid * BPW

    pltpu.sync_copy(words_hbm.at[:, pl.ds(brow0, BPW)], idx_v)

    def fire_g1(l, b):
        pltpu.async_copy(pre_hbm.at[idx_v.at[l]], bufs[b], semg[b])

    def wait_g1(l, b):
        pltpu.make_async_copy(pre_hbm.at[idx_v.at[l]], bufs[b], semg[b]).wait()

    def fire_g2(l, b):
        pltpu.async_copy(trn_hbm.at[idx_v.at[l]], bufs[b], sema[b], add=True)

    def wait_g2(l, b):
        pltpu.make_async_copy(trn_hbm.at[idx_v.at[l]], bufs[b], sema[b]).wait()

    def fire_out(l, b):
        pltpu.async_copy(bufs[b], out_hbm.at[l, pl.ds(brow0, BPW)], semo[b])

    def wait_out(b):
        pltpu.make_async_copy(bufs[b], out_hbm.at[0, pl.ds(brow0, BPW)], semo[b]).wait()

    for b in range(NBUF):
        fire_g1(b, b)

    def round_body(g, carry):
        l0 = g * NBUF
        for b in range(NBUF):
            wait_g1(l0 + b, b)
            fire_g2(l0 + b, b)
        for b in range(NBUF):
            wait_g2(l0 + b, b)
            fire_out(l0 + b, b)
        for b in range(NBUF):
            wait_out(b)
            fire_g1(l0 + b + NBUF, b)
        return carry

    lax.fori_loop(0, ROUNDS - 1, round_body, 0)

    l0 = (ROUNDS - 1) * NBUF
    for b in range(NBUF):
        wait_g1(l0 + b, b)
        fire_g2(l0 + b, b)
    for b in range(NBUF):
        wait_g2(l0 + b, b)
        fire_out(l0 + b, b)
    for b in range(NBUF):
        wait_out(b)


@jax.jit
def _run(words, pre, trn):
    mesh = plsc.VectorSubcoreMesh(
        core_axis_name="c", subcore_axis_name="s", num_cores=NC, num_subcores=NS
    )
    f = pl.kernel(
        _body,
        out_type=jax.ShapeDtypeStruct((L, B, DIM), jnp.float32),
        mesh=mesh,
        scratch_types=(
            [pltpu.VMEM((L, BPW), jnp.int32)]
            + [pltpu.VMEM((CHUNK, DIM), jnp.float32) for _ in range(NBUF)]

            + [pltpu.SemaphoreType.DMA for _ in range(3 * NBUF)]
        ),
    )
    out_lbd = f(jnp.transpose(words), pre, trn)
    return jnp.transpose(out_lbd, (1, 0, 2))


def kernel(words, pretrained_weight, trainable_weight):
    return _run(words, pretrained_weight, trainable_weight)
